# baseline (speedup 1.0000x reference)
import functools

import jax
import jax.numpy as jnp
from jax import lax
from jax.experimental import pallas as pl
from jax.experimental.pallas import tpu as pltpu
from jax.experimental.pallas import tpu_sc as plsc

_F = 64
_L = 16
_NC = 2
_NS = 16
_NW = _NC * _NS
_IDX_W = 128


def _make_mf(B: int):
    b_per_w = B // _NW
    n_chunks = b_per_w // _IDX_W
    mesh = plsc.VectorSubcoreMesh(core_axis_name="c", subcore_axis_name="s")

    @functools.partial(
        pl.kernel,
        out_type=jax.ShapeDtypeStruct((B,), jnp.float32),
        mesh=mesh,
        scratch_types=[
            pltpu.VMEM((n_chunks, _IDX_W), jnp.int32),
            pltpu.VMEM((n_chunks, _IDX_W), jnp.int32),
            pltpu.VMEM((b_per_w, _F), jnp.float32),
            pltpu.VMEM((b_per_w, _F), jnp.float32),
            pltpu.VMEM((b_per_w,), jnp.float32),
            pltpu.VMEM((_L * _L,), jnp.float32),
            pltpu.SemaphoreType.DMA,
        ],
        compiler_params=pltpu.CompilerParams(
            needs_layout_passes=False, use_tc_tiling_on_sc=False),
    )
    def mf(users_hbm, items_hbm, uf_hbm, if_hbm, out_hbm,
           uidx, iidx, urows, irows, outv, tbuf, sem):
        wid = lax.axis_index("s") * _NC + lax.axis_index("c")
        cbase = wid * n_chunks
        bbase = wid * b_per_w

        pltpu.sync_copy(users_hbm.at[pl.ds(cbase, n_chunks)], uidx)
        pltpu.sync_copy(items_hbm.at[pl.ds(cbase, n_chunks)], iidx)

        copies = []
        for j in range(n_chunks):
            copies.append(pltpu.async_copy(
                uf_hbm.at[uidx.at[j]], urows.at[pl.ds(j * _IDX_W, _IDX_W)],
                sem))
            copies.append(pltpu.async_copy(
                if_hbm.at[iidx.at[j]], irows.at[pl.ds(j * _IDX_W, _IDX_W)],
                sem))
        for c in copies:
            c.wait()

        def group(g, carry):
            for e in range(_L):
                b = g * _L + e
                acc = None
                for c in range(_F // _L):
                    u = urows[b, pl.ds(c * _L, _L)]
                    v = irows[b, pl.ds(c * _L, _L)]
                    uv = u * v
                    acc = uv if acc is None else acc + uv
                tbuf[pl.ds(e * _L, _L)] = acc
            lane = lax.iota(jnp.int32, _L) * _L
            res = jnp.zeros((_L,), jnp.float32)
            for c in range(_L):
                res = res + plsc.load_gather(tbuf, [lane + c])
            outv[pl.ds(g * _L, _L)] = res
            return carry

        lax.fori_loop(0, b_per_w // _L, group, 0)
        pltpu.sync_copy(outv, out_hbm.at[pl.ds(bbase, b_per_w)])

    return mf


def kernel(users, items, user_factors, item_factors):
    B = users.shape[0]
    b_per_w = B // _NW
    n_chunks = b_per_w // _IDX_W
    users2d = users.astype(jnp.int32).reshape(_NW * n_chunks, _IDX_W)
    items2d = items.astype(jnp.int32).reshape(_NW * n_chunks, _IDX_W)
    mf = _make_mf(B)
    return mf(users2d, items2d, user_factors, item_factors)


# speedup vs baseline: 2.3334x; 2.3334x over previous
import functools

import jax
import jax.numpy as jnp
from jax import lax
from jax.experimental import pallas as pl
from jax.experimental.pallas import tpu as pltpu
from jax.experimental.pallas import tpu_sc as plsc

_F = 64
_L = 16
_NC = 2
_NS = 16
_NW = _NC * _NS
_TW = 128
_CH = 4


def _make_mf(B: int):
    b_per_w = B // _NW
    n_groups = b_per_w // _L
    mesh = plsc.VectorSubcoreMesh(core_axis_name="c", subcore_axis_name="s")

    @functools.partial(
        pl.kernel,
        out_type=jax.ShapeDtypeStruct((B,), jnp.float32),
        mesh=mesh,
        scratch_types=[
            pltpu.VMEM((b_per_w,), jnp.int32),
            pltpu.VMEM((b_per_w,), jnp.int32),
            pltpu.VMEM((_CH, _F, _TW), jnp.float32),
            pltpu.VMEM((_CH, _F, _TW), jnp.float32),
            pltpu.VMEM((b_per_w,), jnp.float32),
            pltpu.VMEM((_L * _L,), jnp.float32),
            pltpu.SemaphoreType.DMA,
        ],
        compiler_params=pltpu.CompilerParams(needs_layout_passes=False),
    )
    def mf(users_hbm, items_hbm, uft_hbm, ift_hbm, out_hbm,
           usmem, ismem, ublk, iblk, outv, tbuf, sem):
        wid = lax.axis_index("s") * _NC + lax.axis_index("c")
        bbase = wid * b_per_w

        pltpu.sync_copy(users_hbm.at[pl.ds(bbase, b_per_w)], usmem)
        pltpu.sync_copy(items_hbm.at[pl.ds(bbase, b_per_w)], ismem)

        iota = lax.iota(jnp.int32, _L)

        def group(g, carry):
            uvec = usmem[pl.ds(g * _L, _L)]
            vvec = ismem[pl.ds(g * _L, _L)]
            ublks = (uvec // _TW) * _TW
            vblks = (vvec // _TW) * _TW
            ucols = uvec - ublks
            vcols = vvec - vblks
            for s in range(_L // _CH):
                copies = []
                for e in range(_CH):
                    uoff = pl.multiple_of(ublks[s * _CH + e], _TW)
                    voff = pl.multiple_of(vblks[s * _CH + e], _TW)
                    copies.append(pltpu.async_copy(
                        uft_hbm.at[:, pl.ds(uoff, _TW)],
                        ublk.at[e], sem))
                    copies.append(pltpu.async_copy(
                        ift_hbm.at[:, pl.ds(voff, _TW)],
                        iblk.at[e], sem))
                for c in copies:
                    c.wait()
                for e in range(_CH):
                    uc = jnp.full((_L,), ucols[s * _CH + e], jnp.int32)
                    vc = jnp.full((_L,), vcols[s * _CH + e], jnp.int32)
                    ev = jnp.full((_L,), e, jnp.int32)
                    acc = None
                    for c4 in range(_F // _L):
                        fvec = c4 * _L + iota
                        uvals = plsc.load_gather(ublk, [ev, fvec, uc])
                        ivals = plsc.load_gather(iblk, [ev, fvec, vc])
                        uv = uvals * ivals
                        acc = uv if acc is None else acc + uv
                    tbuf[pl.ds((s * _CH + e) * _L, _L)] = acc

            lane = iota * _L
            res = jnp.zeros((_L,), jnp.float32)
            for c in range(_L):
                res = res + plsc.load_gather(tbuf, [lane + c])
            outv[pl.ds(g * _L, _L)] = res
            return carry

        lax.fori_loop(0, n_groups, group, 0)
        pltpu.sync_copy(outv, out_hbm.at[pl.ds(bbase, b_per_w)])

    return mf


def kernel(users, items, user_factors, item_factors):
    B = users.shape[0]
    mf = _make_mf(B)
    return mf(users.astype(jnp.int32), items.astype(jnp.int32),
              user_factors.T, item_factors.T)
